# Initial kernel scaffold; baseline (speedup 1.0000x reference)
import jax
import jax.numpy as jnp
from jax.experimental import pallas as pl


def kernel(inputs, token_table, position_table):
    raise NotImplementedError("write your pallas kernel here")


# speedup vs baseline: 1.4482x; 1.4482x over previous
import functools

import jax
import jax.numpy as jnp
from jax import lax
from jax.experimental import pallas as pl
from jax.experimental.pallas import tpu as pltpu
from jax.experimental.pallas import tpu_sc as plsc

SEQ_LEN = 200
DIM = 32
BATCH = 4096
B = BATCH * SEQ_LEN
NC = 2
NS = 16
NW = NC * NS
BPW = B // NW
CHUNK = 3200
G = CHUNK // 128
NCHUNK = BPW // CHUNK
REPS = CHUNK // SEQ_LEN
LPR = DIM // 16

_mesh = plsc.VectorSubcoreMesh(core_axis_name="c", subcore_axis_name="s")


@functools.partial(
    pl.kernel,
    mesh=_mesh,
    out_type=jax.ShapeDtypeStruct((B, DIM), jnp.float32),
    scratch_types=[
        pltpu.VMEM((CHUNK,), jnp.int32),
        pltpu.VMEM((CHUNK, DIM), jnp.float32),
        pltpu.VMEM((SEQ_LEN, DIM), jnp.float32),
        pltpu.SemaphoreType.DMA,
    ],
    compiler_params=pltpu.CompilerParams(use_tc_tiling_on_sc=False),
)
def _emb_kernel(idx_hbm, tok_hbm, pos_hbm, out_hbm, idx_v, rows_v, pos_v, sem):
    wid = lax.axis_index("s") * NC + lax.axis_index("c")
    row0 = wid * BPW
    pltpu.sync_copy(pos_hbm, pos_v)

    def chunk_body(c, _):
        base = row0 + c * CHUNK
        pltpu.sync_copy(idx_hbm.at[pl.ds(base, CHUNK)], idx_v)
        pltpu.async_copy(tok_hbm.at[idx_v], rows_v, sem).wait()

        def add_pos(r, _):
            for k in range(LPR):
                pvec = pos_v[r, pl.ds(k * 16, 16)]
                for s in range(REPS):
                    plsc.addupdate(
                        rows_v.at[s * SEQ_LEN + r, pl.ds(k * 16, 16)], pvec
                    )
            return 0

        lax.fori_loop(0, SEQ_LEN, add_pos, 0)

        pltpu.sync_copy(rows_v, out_hbm.at[pl.ds(base, CHUNK)])
        return 0

    lax.fori_loop(0, NCHUNK, chunk_body, 0)


def kernel(inputs, token_table, position_table):
    flat_idx = inputs.reshape(B).astype(jnp.int32)
    out = _emb_kernel(flat_idx, token_table, position_table)
    return out.reshape(BATCH, SEQ_LEN, DIM)


# speedup vs baseline: 1.4856x; 1.0259x over previous
import functools

import jax
import jax.numpy as jnp
from jax import lax
from jax.experimental import pallas as pl
from jax.experimental.pallas import tpu as pltpu
from jax.experimental.pallas import tpu_sc as plsc

SEQ_LEN = 200
DIM = 32
BATCH = 4096
B = BATCH * SEQ_LEN
NC = 2
NS = 16
NW = NC * NS
BPW = B // NW
CHUNK = 1600
NCHUNK = BPW // CHUNK
REPS = CHUNK // SEQ_LEN
LPR = DIM // 16

_mesh = plsc.VectorSubcoreMesh(core_axis_name="c", subcore_axis_name="s")


@functools.partial(
    pl.kernel,
    mesh=_mesh,
    out_type=jax.ShapeDtypeStruct((B, DIM), jnp.float32),
    scratch_types=[
        pltpu.VMEM((CHUNK,), jnp.int32),
        pltpu.VMEM((CHUNK,), jnp.int32),
        pltpu.VMEM((CHUNK, DIM), jnp.float32),
        pltpu.VMEM((CHUNK, DIM), jnp.float32),
        pltpu.VMEM((SEQ_LEN, DIM), jnp.float32),
        pltpu.SemaphoreType.DMA,
        pltpu.SemaphoreType.DMA,
        pltpu.SemaphoreType.DMA,
        pltpu.SemaphoreType.DMA,
    ],
    compiler_params=pltpu.CompilerParams(use_tc_tiling_on_sc=False),
)
def _emb_kernel(idx_hbm, tok_hbm, pos_hbm, out_hbm,
                idx0, idx1, rows0, rows1, pos_v,
                sg0, sg1, so0, so1):
    wid = lax.axis_index("s") * NC + lax.axis_index("c")
    row0 = wid * BPW
    pltpu.sync_copy(pos_hbm, pos_v)

    def launch(c, ib, rb, sg, so):
        @pl.when(c < NCHUNK)
        def _():
            @pl.when(c >= 2)
            def _():
                pltpu.make_async_copy(
                    rb, out_hbm.at[pl.ds(row0, CHUNK)], so
                ).wait()
            pltpu.sync_copy(idx_hbm.at[pl.ds(row0 + c * CHUNK, CHUNK)], ib)
            pltpu.async_copy(tok_hbm.at[ib], rb, sg)

    def process(c, ib, rb, sg, so):
        pltpu.make_async_copy(tok_hbm.at[ib], rb, sg).wait()

        def add_pos(r, _):
            for k in range(LPR):
                pvec = pos_v[r, pl.ds(k * 16, 16)]
                for s in range(REPS):
                    plsc.addupdate(
                        rb.at[s * SEQ_LEN + r, pl.ds(k * 16, 16)], pvec
                    )
            return 0

        lax.fori_loop(0, SEQ_LEN, add_pos, 0)
        pltpu.async_copy(rb, out_hbm.at[pl.ds(row0 + c * CHUNK, CHUNK)], so)

    launch(0, idx0, rows0, sg0, so0)

    def pair_body(t, _):
        g = 2 * t
        launch(g + 1, idx1, rows1, sg1, so1)
        process(g, idx0, rows0, sg0, so0)
        launch(g + 2, idx0, rows0, sg0, so0)
        process(g + 1, idx1, rows1, sg1, so1)
        return 0

    lax.fori_loop(0, NCHUNK // 2, pair_body, 0)

    pltpu.make_async_copy(rows0, out_hbm.at[pl.ds(row0, CHUNK)], so0).wait()
    pltpu.make_async_copy(rows1, out_hbm.at[pl.ds(row0, CHUNK)], so1).wait()


def kernel(inputs, token_table, position_table):
    flat_idx = inputs.reshape(B).astype(jnp.int32)
    out = _emb_kernel(flat_idx, token_table, position_table)
    return out.reshape(BATCH, SEQ_LEN, DIM)
